# Initial kernel scaffold; baseline (speedup 1.0000x reference)
import jax
import jax.numpy as jnp
from jax.experimental import pallas as pl


def kernel(x, rel, rel_ids, adj, W_heads, a1_heads, a2_heads, ar_heads, W_out, a1_out, a2_out, ar_out, lin_W, lin_b):
    raise NotImplementedError("write your pallas kernel here")


# speedup vs baseline: 1.3380x; 1.3380x over previous
import functools

import jax
import jax.numpy as jnp
from jax.experimental import pallas as pl
from jax.experimental.pallas import tpu as pltpu

_ALPHA = 0.2
_MASKV = -9e15
_PADV = -1e30


def _proj_kernel(x_ref, w_ref, a1_ref, a2_ref, ar_ref, relt_ref,
                 h_ref, f1_ref, f2_ref, sc_ref, *, n, npad):
    h = jnp.dot(x_ref[...], w_ref[0], preferred_element_type=jnp.float32)
    h_ref[0, :n, :] = h
    f1_ref[0, :n, :] = jnp.dot(h, a1_ref[0], preferred_element_type=jnp.float32)
    f2_ref[0, :n, :] = jnp.dot(h, a2_ref[0], preferred_element_type=jnp.float32)
    if npad > n:
        h_ref[0, n:, :] = jnp.zeros((npad - n, h.shape[1]), jnp.float32)
        f1_ref[0, n:, :] = jnp.zeros((npad - n, 1), jnp.float32)
        f2_ref[0, n:, :] = jnp.zeros((npad - n, 1), jnp.float32)
    sc_ref[...] = jnp.dot(ar_ref[...], relt_ref[...],
                          preferred_element_type=jnp.float32)


def _flash_step(hh, sc_ref, ids, sel, base, f1h, f2h, hb, m_scr, l_scr,
                acc_scr, nrel):
    bias = jnp.zeros(ids.shape, jnp.float32)
    for r in range(nrel):
        ind = (ids == r).astype(jnp.float32)
        bias = bias + ind * sc_ref[hh, r]
    ee = f1h + f2h + bias
    ee = jnp.where(ee >= 0, ee, _ALPHA * ee)
    e = jnp.where(sel, ee, base)
    m_old = m_scr[hh]
    l_old = l_scr[hh]
    m_new = jnp.maximum(m_old, jnp.max(e, axis=1, keepdims=True))
    p = jnp.exp(e - m_new)
    corr = jnp.exp(m_old - m_new)
    acc_scr[hh] = acc_scr[hh] * corr + jnp.dot(
        p, hb, preferred_element_type=jnp.float32)
    l_scr[hh] = l_old * corr + jnp.sum(p, axis=1, keepdims=True)
    m_scr[hh] = m_new


def _attn1_kernel(adj_ref, ids_ref, h_ref, f1_ref, f2_ref, sc_ref,
                  wout_ref, a1o_ref, a2o_ref,
                  h2_ref, f12_ref, f22_ref,
                  m_scr, l_scr, acc_scr, *, n, nheads, nrel, br, bc, nc):
    i = pl.program_id(0)
    j = pl.program_id(1)

    @pl.when(j == 0)
    def _():
        m_scr[...] = jnp.full(m_scr.shape, -jnp.inf, jnp.float32)
        l_scr[...] = jnp.zeros(l_scr.shape, jnp.float32)
        acc_scr[...] = jnp.zeros(acc_scr.shape, jnp.float32)

    ids = ids_ref[...]
    col = j * bc + jax.lax.broadcasted_iota(jnp.int32, (1, bc), 1)
    col_ok = col < n
    base = jnp.where(col_ok, _MASKV, _PADV)
    sel = (adj_ref[...] > 0) & col_ok

    for hh in range(nheads):
        hb = h_ref[hh, pl.ds(j * bc, bc), :]
        _flash_step(hh, sc_ref, ids, sel, base,
                    f1_ref[hh], f2_ref[hh], hb, m_scr, l_scr, acc_scr, nrel)

    @pl.when(j == nc - 1)
    def _():
        xm = jnp.zeros(acc_scr.shape[1:], jnp.float32)
        for hh in range(nheads):
            o = acc_scr[hh] / l_scr[hh]
            xm = xm + jnp.where(o > 0, o, jnp.exp(o) - 1.0)
        xm = xm * (1.0 / nheads)
        row = i * br + jax.lax.broadcasted_iota(jnp.int32, (br, 1), 0)
        xm = jnp.where(row < n, xm, 0.0)
        h2 = jnp.dot(xm, wout_ref[...], preferred_element_type=jnp.float32)
        h2_ref[...] = h2
        f12_ref[...] = jnp.dot(h2, a1o_ref[...],
                               preferred_element_type=jnp.float32)
        f22_ref[...] = jnp.dot(h2, a2o_ref[...],
                               preferred_element_type=jnp.float32)


def _attn2_kernel(adj_ref, ids_ref, h_ref, f1_ref, f2_ref, sc_ref,
                  linw_ref, linb_ref, out_ref,
                  m_scr, l_scr, acc_scr, *, n, nrel, br, bc, nc, sc_row):
    j = pl.program_id(1)

    @pl.when(j == 0)
    def _():
        m_scr[...] = jnp.full(m_scr.shape, -jnp.inf, jnp.float32)
        l_scr[...] = jnp.zeros(l_scr.shape, jnp.float32)
        acc_scr[...] = jnp.zeros(acc_scr.shape, jnp.float32)

    ids = ids_ref[...]
    col = j * bc + jax.lax.broadcasted_iota(jnp.int32, (1, bc), 1)
    col_ok = col < n
    base = jnp.where(col_ok, _MASKV, _PADV)
    sel = (adj_ref[...] > 0) & col_ok

    hb = h_ref[pl.ds(j * bc, bc), :]
    bias = jnp.zeros(ids.shape, jnp.float32)
    for r in range(nrel):
        ind = (ids == r).astype(jnp.float32)
        bias = bias + ind * sc_ref[sc_row, r]
    ee = f1_ref[...] + f2_ref[...] + bias
    ee = jnp.where(ee >= 0, ee, _ALPHA * ee)
    e = jnp.where(sel, ee, base)
    m_old = m_scr[0]
    l_old = l_scr[0]
    m_new = jnp.maximum(m_old, jnp.max(e, axis=1, keepdims=True))
    p = jnp.exp(e - m_new)
    corr = jnp.exp(m_old - m_new)
    acc_scr[0] = acc_scr[0] * corr + jnp.dot(
        p, hb, preferred_element_type=jnp.float32)
    l_scr[0] = l_old * corr + jnp.sum(p, axis=1, keepdims=True)
    m_scr[0] = m_new

    @pl.when(j == nc - 1)
    def _():
        xo = acc_scr[0] / l_scr[0]
        logits = jnp.dot(xo, linw_ref[...],
                         preferred_element_type=jnp.float32) + linb_ref[...]
        z = logits - jnp.max(logits, axis=1, keepdims=True)
        out_ref[...] = z - jnp.log(jnp.sum(jnp.exp(z), axis=1, keepdims=True))


def kernel(x, rel, rel_ids, adj, W_heads, a1_heads, a2_heads, ar_heads,
           W_out, a1_out, a2_out, ar_out, lin_W, lin_b):
    n, f = x.shape
    nheads = W_heads.shape[0]
    nrel = rel.shape[0]
    nclass = lin_W.shape[1]

    br, bc = 256, 512
    npad = -(-n // bc) * bc
    nr = npad // br
    nc = npad // bc
    sc_rows = 8

    a1c = a1_heads[:, :, None]
    a2c = a2_heads[:, :, None]
    ar_all = jnp.zeros((sc_rows, f), jnp.float32)
    ar_all = ar_all.at[:nheads].set(ar_heads).at[nheads].set(ar_out)
    relt = rel.T

    h1, f1c, f2c, scores = pl.pallas_call(
        functools.partial(_proj_kernel, n=n, npad=npad),
        grid=(nheads,),
        in_specs=[
            pl.BlockSpec((n, f), lambda h: (0, 0)),
            pl.BlockSpec((1, f, f), lambda h: (h, 0, 0)),
            pl.BlockSpec((1, f, 1), lambda h: (h, 0, 0)),
            pl.BlockSpec((1, f, 1), lambda h: (h, 0, 0)),
            pl.BlockSpec((sc_rows, f), lambda h: (0, 0)),
            pl.BlockSpec((f, nrel), lambda h: (0, 0)),
        ],
        out_specs=[
            pl.BlockSpec((1, npad, f), lambda h: (h, 0, 0)),
            pl.BlockSpec((1, npad, 1), lambda h: (h, 0, 0)),
            pl.BlockSpec((1, npad, 1), lambda h: (h, 0, 0)),
            pl.BlockSpec((sc_rows, nrel), lambda h: (0, 0)),
        ],
        out_shape=[
            jax.ShapeDtypeStruct((nheads, npad, f), jnp.float32),
            jax.ShapeDtypeStruct((nheads, npad, 1), jnp.float32),
            jax.ShapeDtypeStruct((nheads, npad, 1), jnp.float32),
            jax.ShapeDtypeStruct((sc_rows, nrel), jnp.float32),
        ],
    )(x, W_heads, a1c, a2c, ar_all, relt)

    f2r = f2c.transpose(0, 2, 1)

    h2, f12c, f22c = pl.pallas_call(
        functools.partial(_attn1_kernel, n=n, nheads=nheads, nrel=nrel,
                          br=br, bc=bc, nc=nc),
        grid=(nr, nc),
        in_specs=[
            pl.BlockSpec((br, bc), lambda i, j: (i, j)),
            pl.BlockSpec((br, bc), lambda i, j: (i, j)),
            pl.BlockSpec((nheads, npad, f), lambda i, j: (0, 0, 0)),
            pl.BlockSpec((nheads, br, 1), lambda i, j: (0, i, 0)),
            pl.BlockSpec((nheads, 1, bc), lambda i, j: (0, 0, j)),
            pl.BlockSpec((sc_rows, nrel), lambda i, j: (0, 0)),
            pl.BlockSpec((f, f), lambda i, j: (0, 0)),
            pl.BlockSpec((f, 1), lambda i, j: (0, 0)),
            pl.BlockSpec((f, 1), lambda i, j: (0, 0)),
        ],
        out_specs=[
            pl.BlockSpec((br, f), lambda i, j: (i, 0)),
            pl.BlockSpec((br, 1), lambda i, j: (i, 0)),
            pl.BlockSpec((br, 1), lambda i, j: (i, 0)),
        ],
        out_shape=[
            jax.ShapeDtypeStruct((npad, f), jnp.float32),
            jax.ShapeDtypeStruct((npad, 1), jnp.float32),
            jax.ShapeDtypeStruct((npad, 1), jnp.float32),
        ],
        scratch_shapes=[
            pltpu.VMEM((nheads, br, 1), jnp.float32),
            pltpu.VMEM((nheads, br, 1), jnp.float32),
            pltpu.VMEM((nheads, br, f), jnp.float32),
        ],
        compiler_params=pltpu.CompilerParams(
            dimension_semantics=("parallel", "arbitrary")),
    )(adj, rel_ids, h1, f1c, f2r, scores, W_out,
      a1_out[:, None], a2_out[:, None])

    f22r = f22c.T

    out = pl.pallas_call(
        functools.partial(_attn2_kernel, n=n, nrel=nrel, br=br, bc=bc,
                          nc=nc, sc_row=nheads),
        grid=(nr, nc),
        in_specs=[
            pl.BlockSpec((br, bc), lambda i, j: (i, j)),
            pl.BlockSpec((br, bc), lambda i, j: (i, j)),
            pl.BlockSpec((npad, f), lambda i, j: (0, 0)),
            pl.BlockSpec((br, 1), lambda i, j: (i, 0)),
            pl.BlockSpec((1, bc), lambda i, j: (0, j)),
            pl.BlockSpec((sc_rows, nrel), lambda i, j: (0, 0)),
            pl.BlockSpec((f, nclass), lambda i, j: (0, 0)),
            pl.BlockSpec((1, nclass), lambda i, j: (0, 0)),
        ],
        out_specs=pl.BlockSpec((br, nclass), lambda i, j: (i, 0)),
        out_shape=jax.ShapeDtypeStruct((npad, nclass), jnp.float32),
        scratch_shapes=[
            pltpu.VMEM((1, br, 1), jnp.float32),
            pltpu.VMEM((1, br, 1), jnp.float32),
            pltpu.VMEM((1, br, f), jnp.float32),
        ],
        compiler_params=pltpu.CompilerParams(
            dimension_semantics=("parallel", "arbitrary")),
    )(adj, rel_ids, h2, f12c, f22r, scores, lin_W, lin_b[None, :])

    return out[:n]


# speedup vs baseline: 1.7535x; 1.3105x over previous
import functools

import jax
import jax.numpy as jnp
from jax.experimental import pallas as pl
from jax.experimental.pallas import tpu as pltpu

_ALPHA = 0.2
_MASKV = -9e15
_PADV = -1e30


def _proj_kernel(x_ref, w_ref, a1_ref, a2_ref, ar_ref, relt_ref,
                 h_ref, f1_ref, f2_ref, sc_ref, *, n, npad):
    h = jnp.dot(x_ref[...], w_ref[0], preferred_element_type=jnp.float32)
    h_ref[0, :n, :] = h
    f1_ref[0, :n, :] = jnp.dot(h, a1_ref[0], preferred_element_type=jnp.float32)
    f2_ref[0, :n, :] = jnp.dot(h, a2_ref[0], preferred_element_type=jnp.float32)
    if npad > n:
        h_ref[0, n:, :] = jnp.zeros((npad - n, h.shape[1]), jnp.float32)
        f1_ref[0, n:, :] = jnp.zeros((npad - n, 1), jnp.float32)
        f2_ref[0, n:, :] = jnp.zeros((npad - n, 1), jnp.float32)
    sc_ref[...] = jnp.dot(ar_ref[...], relt_ref[...],
                          preferred_element_type=jnp.float32)


def _bias_lookup(sc_ref, row, bits, ids, nrel):
    if bits is not None:
        vals = [sc_ref[row, r] for r in range(nrel)]
        lvl = 0
        while len(vals) > 1:
            vals = [jnp.where(bits[lvl], vals[2 * k + 1], vals[2 * k])
                    for k in range(len(vals) // 2)]
            lvl += 1
        return vals[0]
    bias = jnp.zeros(ids.shape, jnp.float32)
    for r in range(nrel):
        bias = bias + (ids == r).astype(jnp.float32) * sc_ref[row, r]
    return bias


def _rel_bits(ids, nrel):
    nbits = nrel.bit_length() - 1
    if (1 << nbits) != nrel:
        return None
    return [(ids & (1 << k)) != 0 for k in range(nbits)]


def _flash_step(hh, sc_ref, ids, bits, sel, base, f1h, f2h, hb, m_scr, l_scr,
                acc_scr, nrel):
    bias = _bias_lookup(sc_ref, hh, bits, ids, nrel)
    ee = f1h + f2h + bias
    ee = jnp.maximum(ee, _ALPHA * ee)
    e = jnp.where(sel, ee, base)
    m_old = m_scr[hh]
    l_old = l_scr[hh]
    m_new = jnp.maximum(m_old, jnp.max(e, axis=1, keepdims=True))
    p = jnp.exp(e - m_new)
    corr = jnp.exp(m_old - m_new)
    acc_scr[hh] = acc_scr[hh] * corr + jnp.dot(
        p, hb, preferred_element_type=jnp.float32)
    l_scr[hh] = l_old * corr + jnp.sum(p, axis=1, keepdims=True)
    m_scr[hh] = m_new


def _attn1_kernel(adj_ref, ids_ref, h_ref, f1_ref, f2_ref, sc_ref,
                  wout_ref, a1o_ref, a2o_ref,
                  h2_ref, f12_ref, f22_ref,
                  m_scr, l_scr, acc_scr, *, n, nheads, nrel, br, bc, nc):
    i = pl.program_id(0)
    j = pl.program_id(1)

    @pl.when(j == 0)
    def _():
        m_scr[...] = jnp.full(m_scr.shape, -jnp.inf, jnp.float32)
        l_scr[...] = jnp.zeros(l_scr.shape, jnp.float32)
        acc_scr[...] = jnp.zeros(acc_scr.shape, jnp.float32)

    ids = ids_ref[...]
    col = j * bc + jax.lax.broadcasted_iota(jnp.int32, (1, bc), 1)
    col_ok = col < n
    base = jnp.where(col_ok, _MASKV, _PADV)
    sel = (adj_ref[...] > 0) & col_ok
    bits = _rel_bits(ids, nrel)

    for hh in range(nheads):
        hb = h_ref[hh, pl.ds(j * bc, bc), :]
        _flash_step(hh, sc_ref, ids, bits, sel, base,
                    f1_ref[hh], f2_ref[hh], hb, m_scr, l_scr, acc_scr, nrel)

    @pl.when(j == nc - 1)
    def _():
        xm = jnp.zeros(acc_scr.shape[1:], jnp.float32)
        for hh in range(nheads):
            o = acc_scr[hh] / l_scr[hh]
            xm = xm + jnp.where(o > 0, o, jnp.exp(o) - 1.0)
        xm = xm * (1.0 / nheads)
        row = i * br + jax.lax.broadcasted_iota(jnp.int32, (br, 1), 0)
        xm = jnp.where(row < n, xm, 0.0)
        h2 = jnp.dot(xm, wout_ref[...], preferred_element_type=jnp.float32)
        h2_ref[...] = h2
        f12_ref[...] = jnp.dot(h2, a1o_ref[...],
                               preferred_element_type=jnp.float32)
        f22_ref[...] = jnp.dot(h2, a2o_ref[...],
                               preferred_element_type=jnp.float32)


def _attn2_kernel(adj_ref, ids_ref, h_ref, f1_ref, f2_ref, sc_ref,
                  linw_ref, linb_ref, out_ref,
                  m_scr, l_scr, acc_scr, *, n, nrel, br, bc, nc, sc_row):
    j = pl.program_id(1)

    @pl.when(j == 0)
    def _():
        m_scr[...] = jnp.full(m_scr.shape, -jnp.inf, jnp.float32)
        l_scr[...] = jnp.zeros(l_scr.shape, jnp.float32)
        acc_scr[...] = jnp.zeros(acc_scr.shape, jnp.float32)

    ids = ids_ref[...]
    col = j * bc + jax.lax.broadcasted_iota(jnp.int32, (1, bc), 1)
    col_ok = col < n
    base = jnp.where(col_ok, _MASKV, _PADV)
    sel = (adj_ref[...] > 0) & col_ok

    hb = h_ref[pl.ds(j * bc, bc), :]
    bias = _bias_lookup(sc_ref, sc_row, _rel_bits(ids, nrel), ids, nrel)
    ee = f1_ref[...] + f2_ref[...] + bias
    ee = jnp.maximum(ee, _ALPHA * ee)
    e = jnp.where(sel, ee, base)
    m_old = m_scr[0]
    l_old = l_scr[0]
    m_new = jnp.maximum(m_old, jnp.max(e, axis=1, keepdims=True))
    p = jnp.exp(e - m_new)
    corr = jnp.exp(m_old - m_new)
    acc_scr[0] = acc_scr[0] * corr + jnp.dot(
        p, hb, preferred_element_type=jnp.float32)
    l_scr[0] = l_old * corr + jnp.sum(p, axis=1, keepdims=True)
    m_scr[0] = m_new

    @pl.when(j == nc - 1)
    def _():
        xo = acc_scr[0] / l_scr[0]
        logits = jnp.dot(xo, linw_ref[...],
                         preferred_element_type=jnp.float32) + linb_ref[...]
        z = logits - jnp.max(logits, axis=1, keepdims=True)
        out_ref[...] = z - jnp.log(jnp.sum(jnp.exp(z), axis=1, keepdims=True))


def kernel(x, rel, rel_ids, adj, W_heads, a1_heads, a2_heads, ar_heads,
           W_out, a1_out, a2_out, ar_out, lin_W, lin_b):
    n, f = x.shape
    nheads = W_heads.shape[0]
    nrel = rel.shape[0]
    nclass = lin_W.shape[1]

    br, bc = 256, 512
    npad = -(-n // bc) * bc
    nr = npad // br
    nc = npad // bc
    sc_rows = 8

    a1c = a1_heads[:, :, None]
    a2c = a2_heads[:, :, None]
    ar_all = jnp.zeros((sc_rows, f), jnp.float32)
    ar_all = ar_all.at[:nheads].set(ar_heads).at[nheads].set(ar_out)
    relt = rel.T

    h1, f1c, f2c, scores = pl.pallas_call(
        functools.partial(_proj_kernel, n=n, npad=npad),
        grid=(nheads,),
        in_specs=[
            pl.BlockSpec((n, f), lambda h: (0, 0)),
            pl.BlockSpec((1, f, f), lambda h: (h, 0, 0)),
            pl.BlockSpec((1, f, 1), lambda h: (h, 0, 0)),
            pl.BlockSpec((1, f, 1), lambda h: (h, 0, 0)),
            pl.BlockSpec((sc_rows, f), lambda h: (0, 0)),
            pl.BlockSpec((f, nrel), lambda h: (0, 0)),
        ],
        out_specs=[
            pl.BlockSpec((1, npad, f), lambda h: (h, 0, 0)),
            pl.BlockSpec((1, npad, 1), lambda h: (h, 0, 0)),
            pl.BlockSpec((1, npad, 1), lambda h: (h, 0, 0)),
            pl.BlockSpec((sc_rows, nrel), lambda h: (0, 0)),
        ],
        out_shape=[
            jax.ShapeDtypeStruct((nheads, npad, f), jnp.float32),
            jax.ShapeDtypeStruct((nheads, npad, 1), jnp.float32),
            jax.ShapeDtypeStruct((nheads, npad, 1), jnp.float32),
            jax.ShapeDtypeStruct((sc_rows, nrel), jnp.float32),
        ],
    )(x, W_heads, a1c, a2c, ar_all, relt)

    f2r = f2c.transpose(0, 2, 1)

    h2, f12c, f22c = pl.pallas_call(
        functools.partial(_attn1_kernel, n=n, nheads=nheads, nrel=nrel,
                          br=br, bc=bc, nc=nc),
        grid=(nr, nc),
        in_specs=[
            pl.BlockSpec((br, bc), lambda i, j: (i, j)),
            pl.BlockSpec((br, bc), lambda i, j: (i, j)),
            pl.BlockSpec((nheads, npad, f), lambda i, j: (0, 0, 0)),
            pl.BlockSpec((nheads, br, 1), lambda i, j: (0, i, 0)),
            pl.BlockSpec((nheads, 1, bc), lambda i, j: (0, 0, j)),
            pl.BlockSpec((sc_rows, nrel), lambda i, j: (0, 0)),
            pl.BlockSpec((f, f), lambda i, j: (0, 0)),
            pl.BlockSpec((f, 1), lambda i, j: (0, 0)),
            pl.BlockSpec((f, 1), lambda i, j: (0, 0)),
        ],
        out_specs=[
            pl.BlockSpec((br, f), lambda i, j: (i, 0)),
            pl.BlockSpec((br, 1), lambda i, j: (i, 0)),
            pl.BlockSpec((br, 1), lambda i, j: (i, 0)),
        ],
        out_shape=[
            jax.ShapeDtypeStruct((npad, f), jnp.float32),
            jax.ShapeDtypeStruct((npad, 1), jnp.float32),
            jax.ShapeDtypeStruct((npad, 1), jnp.float32),
        ],
        scratch_shapes=[
            pltpu.VMEM((nheads, br, 1), jnp.float32),
            pltpu.VMEM((nheads, br, 1), jnp.float32),
            pltpu.VMEM((nheads, br, f), jnp.float32),
        ],
        compiler_params=pltpu.CompilerParams(
            dimension_semantics=("parallel", "arbitrary")),
    )(adj, rel_ids, h1, f1c, f2r, scores, W_out,
      a1_out[:, None], a2_out[:, None])

    f22r = f22c.T

    out = pl.pallas_call(
        functools.partial(_attn2_kernel, n=n, nrel=nrel, br=br, bc=bc,
                          nc=nc, sc_row=nheads),
        grid=(nr, nc),
        in_specs=[
            pl.BlockSpec((br, bc), lambda i, j: (i, j)),
            pl.BlockSpec((br, bc), lambda i, j: (i, j)),
            pl.BlockSpec((npad, f), lambda i, j: (0, 0)),
            pl.BlockSpec((br, 1), lambda i, j: (i, 0)),
            pl.BlockSpec((1, bc), lambda i, j: (0, j)),
            pl.BlockSpec((sc_rows, nrel), lambda i, j: (0, 0)),
            pl.BlockSpec((f, nclass), lambda i, j: (0, 0)),
            pl.BlockSpec((1, nclass), lambda i, j: (0, 0)),
        ],
        out_specs=pl.BlockSpec((br, nclass), lambda i, j: (i, 0)),
        out_shape=jax.ShapeDtypeStruct((npad, nclass), jnp.float32),
        scratch_shapes=[
            pltpu.VMEM((1, br, 1), jnp.float32),
            pltpu.VMEM((1, br, 1), jnp.float32),
            pltpu.VMEM((1, br, f), jnp.float32),
        ],
        compiler_params=pltpu.CompilerParams(
            dimension_semantics=("parallel", "arbitrary")),
    )(adj, rel_ids, h2, f12c, f22r, scores, lin_W, lin_b[None, :])

    return out[:n]


# speedup vs baseline: 1.9413x; 1.1071x over previous
import functools

import jax
import jax.numpy as jnp
from jax.experimental import pallas as pl
from jax.experimental.pallas import tpu as pltpu

_ALPHA = 0.2
_MASKV = -9e15
_PADV = -1e30


def _proj_kernel(x_ref, w_ref, a1_ref, a2_ref, ar_ref, relt_ref,
                 h_ref, f1_ref, f2_ref, sc_ref, *, n, npad):
    h = jnp.dot(x_ref[...], w_ref[0], preferred_element_type=jnp.float32)
    h_ref[0, :n, :] = h
    f1_ref[0, :n, :] = jnp.dot(h, a1_ref[0], preferred_element_type=jnp.float32)
    f2_ref[0, :n, :] = jnp.dot(h, a2_ref[0], preferred_element_type=jnp.float32)
    if npad > n:
        h_ref[0, n:, :] = jnp.zeros((npad - n, h.shape[1]), jnp.float32)
        f1_ref[0, n:, :] = jnp.zeros((npad - n, 1), jnp.float32)
        f2_ref[0, n:, :] = jnp.zeros((npad - n, 1), jnp.float32)
    sc_ref[...] = jnp.dot(ar_ref[...], relt_ref[...],
                          preferred_element_type=jnp.float32)


def _tree_sel(vals, bits):
    lvl = 0
    while len(vals) > 1:
        vals = [jnp.where(bits[lvl], vals[2 * k + 1], vals[2 * k])
                for k in range(len(vals) // 2)]
        lvl += 1
    return vals[0]


def _bias_lookup(sc_ref, row, bits, ids, nrel):
    if bits is not None:
        return _tree_sel([sc_ref[row, r] for r in range(nrel)], bits)
    bias = jnp.zeros(ids.shape, jnp.float32)
    for r in range(nrel):
        bias = bias + (ids == r).astype(jnp.float32) * sc_ref[row, r]
    return bias


def _rel_bits(ids, nrel):
    nbits = nrel.bit_length() - 1
    if (1 << nbits) != nrel:
        return None
    return [(ids & (1 << k)) != 0 for k in range(nbits)]


def _flash_step(hh, bias, sel, base, f1h, f2h, hb, m_scr, l_scr, acc_scr):
    ee = f1h + f2h + bias
    ee = jnp.maximum(ee, _ALPHA * ee)
    e = jnp.where(sel, ee, base)
    m_old = m_scr[hh]
    l_old = l_scr[hh]
    m_new = jnp.maximum(m_old, jnp.max(e, axis=1, keepdims=True))
    p = jnp.exp(e - m_new)
    corr = jnp.exp(m_old - m_new)
    acc_scr[hh] = acc_scr[hh] * corr + jnp.dot(
        p, hb, preferred_element_type=jnp.float32)
    l_scr[hh] = l_old * corr + jnp.sum(p, axis=1, keepdims=True)
    m_scr[hh] = m_new


def _attn1_kernel(adj_ref, ids_ref, h_ref, f1_ref, f2_ref, pk_ref,
                  wout_ref, a1o_ref, a2o_ref,
                  h2_ref, f12_ref, f22_ref,
                  m_scr, l_scr, acc_scr, *, n, nheads, nrel, br, bc, nc):
    i = pl.program_id(0)
    j = pl.program_id(1)

    @pl.when(j == 0)
    def _():
        m_scr[...] = jnp.full(m_scr.shape, -jnp.inf, jnp.float32)
        l_scr[...] = jnp.zeros(l_scr.shape, jnp.float32)
        acc_scr[...] = jnp.zeros(acc_scr.shape, jnp.float32)

    ids = ids_ref[...]
    col = j * bc + jax.lax.broadcasted_iota(jnp.int32, (1, bc), 1)
    col_ok = col < n
    base = jnp.where(col_ok, _MASKV, _PADV)
    sel = (adj_ref[...] > 0) & col_ok
    bits = _rel_bits(ids, nrel)

    for pr in range(nheads // 2):
        pv = _tree_sel([pk_ref[pr, r] for r in range(nrel)], bits)
        b_hi = jax.lax.bitcast_convert_type(
            pv & jnp.int32(-65536), jnp.float32)
        b_lo = jax.lax.bitcast_convert_type(pv << 16, jnp.float32)
        for hh, bias in ((2 * pr, b_hi), (2 * pr + 1, b_lo)):
            hb = h_ref[hh, pl.ds(j * bc, bc), :]
            _flash_step(hh, bias, sel, base,
                        f1_ref[hh], f2_ref[hh], hb, m_scr, l_scr, acc_scr)

    @pl.when(j == nc - 1)
    def _():
        xm = jnp.zeros(acc_scr.shape[1:], jnp.float32)
        for hh in range(nheads):
            o = acc_scr[hh] / l_scr[hh]
            xm = xm + jnp.where(o > 0, o, jnp.exp(o) - 1.0)
        xm = xm * (1.0 / nheads)
        row = i * br + jax.lax.broadcasted_iota(jnp.int32, (br, 1), 0)
        xm = jnp.where(row < n, xm, 0.0)
        h2 = jnp.dot(xm, wout_ref[...], preferred_element_type=jnp.float32)
        h2_ref[...] = h2
        f12_ref[...] = jnp.dot(h2, a1o_ref[...],
                               preferred_element_type=jnp.float32)
        f22_ref[...] = jnp.dot(h2, a2o_ref[...],
                               preferred_element_type=jnp.float32)


def _attn2_kernel(adj_ref, ids_ref, h_ref, f1_ref, f2_ref, sc_ref,
                  linw_ref, linb_ref, out_ref,
                  m_scr, l_scr, acc_scr, *, n, nrel, br, bc, nc, sc_row):
    j = pl.program_id(1)

    @pl.when(j == 0)
    def _():
        m_scr[...] = jnp.full(m_scr.shape, -jnp.inf, jnp.float32)
        l_scr[...] = jnp.zeros(l_scr.shape, jnp.float32)
        acc_scr[...] = jnp.zeros(acc_scr.shape, jnp.float32)

    ids = ids_ref[...]
    col = j * bc + jax.lax.broadcasted_iota(jnp.int32, (1, bc), 1)
    col_ok = col < n
    base = jnp.where(col_ok, _MASKV, _PADV)
    sel = (adj_ref[...] > 0) & col_ok

    hb = h_ref[pl.ds(j * bc, bc), :]
    bias = _bias_lookup(sc_ref, sc_row, _rel_bits(ids, nrel), ids, nrel)
    ee = f1_ref[...] + f2_ref[...] + bias
    ee = jnp.maximum(ee, _ALPHA * ee)
    e = jnp.where(sel, ee, base)
    m_old = m_scr[0]
    l_old = l_scr[0]
    m_new = jnp.maximum(m_old, jnp.max(e, axis=1, keepdims=True))
    p = jnp.exp(e - m_new)
    corr = jnp.exp(m_old - m_new)
    acc_scr[0] = acc_scr[0] * corr + jnp.dot(
        p, hb, preferred_element_type=jnp.float32)
    l_scr[0] = l_old * corr + jnp.sum(p, axis=1, keepdims=True)
    m_scr[0] = m_new

    @pl.when(j == nc - 1)
    def _():
        xo = acc_scr[0] / l_scr[0]
        logits = jnp.dot(xo, linw_ref[...],
                         preferred_element_type=jnp.float32) + linb_ref[...]
        z = logits - jnp.max(logits, axis=1, keepdims=True)
        out_ref[...] = z - jnp.log(jnp.sum(jnp.exp(z), axis=1, keepdims=True))


def kernel(x, rel, rel_ids, adj, W_heads, a1_heads, a2_heads, ar_heads,
           W_out, a1_out, a2_out, ar_out, lin_W, lin_b):
    n, f = x.shape
    nheads = W_heads.shape[0]
    nrel = rel.shape[0]
    nclass = lin_W.shape[1]

    br, bc = 256, 512
    npad = -(-n // bc) * bc
    nr = npad // br
    nc = npad // bc
    sc_rows = 8

    a1c = a1_heads[:, :, None]
    a2c = a2_heads[:, :, None]
    ar_all = jnp.zeros((sc_rows, f), jnp.float32)
    ar_all = ar_all.at[:nheads].set(ar_heads).at[nheads].set(ar_out)
    relt = rel.T

    h1, f1c, f2c, scores = pl.pallas_call(
        functools.partial(_proj_kernel, n=n, npad=npad),
        grid=(nheads,),
        in_specs=[
            pl.BlockSpec((n, f), lambda h: (0, 0)),
            pl.BlockSpec((1, f, f), lambda h: (h, 0, 0)),
            pl.BlockSpec((1, f, 1), lambda h: (h, 0, 0)),
            pl.BlockSpec((1, f, 1), lambda h: (h, 0, 0)),
            pl.BlockSpec((sc_rows, f), lambda h: (0, 0)),
            pl.BlockSpec((f, nrel), lambda h: (0, 0)),
        ],
        out_specs=[
            pl.BlockSpec((1, npad, f), lambda h: (h, 0, 0)),
            pl.BlockSpec((1, npad, 1), lambda h: (h, 0, 0)),
            pl.BlockSpec((1, npad, 1), lambda h: (h, 0, 0)),
            pl.BlockSpec((sc_rows, nrel), lambda h: (0, 0)),
        ],
        out_shape=[
            jax.ShapeDtypeStruct((nheads, npad, f), jnp.float32),
            jax.ShapeDtypeStruct((nheads, npad, 1), jnp.float32),
            jax.ShapeDtypeStruct((nheads, npad, 1), jnp.float32),
            jax.ShapeDtypeStruct((sc_rows, nrel), jnp.float32),
        ],
    )(x, W_heads, a1c, a2c, ar_all, relt)

    f2r = f2c.transpose(0, 2, 1)

    sbits = jax.lax.bitcast_convert_type(
        scores[:nheads].astype(jnp.bfloat16), jnp.uint16).astype(jnp.uint32)
    packed = ((sbits[0::2] << 16) | sbits[1::2]).astype(jnp.int32)

    h2, f12c, f22c = pl.pallas_call(
        functools.partial(_attn1_kernel, n=n, nheads=nheads, nrel=nrel,
                          br=br, bc=bc, nc=nc),
        grid=(nr, nc),
        in_specs=[
            pl.BlockSpec((br, bc), lambda i, j: (i, j)),
            pl.BlockSpec((br, bc), lambda i, j: (i, j)),
            pl.BlockSpec((nheads, npad, f), lambda i, j: (0, 0, 0)),
            pl.BlockSpec((nheads, br, 1), lambda i, j: (0, i, 0)),
            pl.BlockSpec((nheads, 1, bc), lambda i, j: (0, 0, j)),
            pl.BlockSpec((nheads // 2, nrel), lambda i, j: (0, 0)),
            pl.BlockSpec((f, f), lambda i, j: (0, 0)),
            pl.BlockSpec((f, 1), lambda i, j: (0, 0)),
            pl.BlockSpec((f, 1), lambda i, j: (0, 0)),
        ],
        out_specs=[
            pl.BlockSpec((br, f), lambda i, j: (i, 0)),
            pl.BlockSpec((br, 1), lambda i, j: (i, 0)),
            pl.BlockSpec((br, 1), lambda i, j: (i, 0)),
        ],
        out_shape=[
            jax.ShapeDtypeStruct((npad, f), jnp.float32),
            jax.ShapeDtypeStruct((npad, 1), jnp.float32),
            jax.ShapeDtypeStruct((npad, 1), jnp.float32),
        ],
        scratch_shapes=[
            pltpu.VMEM((nheads, br, 1), jnp.float32),
            pltpu.VMEM((nheads, br, 1), jnp.float32),
            pltpu.VMEM((nheads, br, f), jnp.float32),
        ],
        compiler_params=pltpu.CompilerParams(
            dimension_semantics=("parallel", "arbitrary")),
    )(adj, rel_ids, h1, f1c, f2r, packed, W_out,
      a1_out[:, None], a2_out[:, None])

    f22r = f22c.T

    out = pl.pallas_call(
        functools.partial(_attn2_kernel, n=n, nrel=nrel, br=br, bc=bc,
                          nc=nc, sc_row=nheads),
        grid=(nr, nc),
        in_specs=[
            pl.BlockSpec((br, bc), lambda i, j: (i, j)),
            pl.BlockSpec((br, bc), lambda i, j: (i, j)),
            pl.BlockSpec((npad, f), lambda i, j: (0, 0)),
            pl.BlockSpec((br, 1), lambda i, j: (i, 0)),
            pl.BlockSpec((1, bc), lambda i, j: (0, j)),
            pl.BlockSpec((sc_rows, nrel), lambda i, j: (0, 0)),
            pl.BlockSpec((f, nclass), lambda i, j: (0, 0)),
            pl.BlockSpec((1, nclass), lambda i, j: (0, 0)),
        ],
        out_specs=pl.BlockSpec((br, nclass), lambda i, j: (i, 0)),
        out_shape=jax.ShapeDtypeStruct((npad, nclass), jnp.float32),
        scratch_shapes=[
            pltpu.VMEM((1, br, 1), jnp.float32),
            pltpu.VMEM((1, br, 1), jnp.float32),
            pltpu.VMEM((1, br, f), jnp.float32),
        ],
        compiler_params=pltpu.CompilerParams(
            dimension_semantics=("parallel", "arbitrary")),
    )(adj, rel_ids, h2, f12c, f22r, scores, lin_W, lin_b[None, :])

    return out[:n]
